# baseline (speedup 1.0000x reference)
import jax
import jax.numpy as jnp
from jax.experimental import pallas as pl
from jax.experimental.pallas import tpu as pltpu

N = 10000
D = 128
BM = 200


def _gcn_body(adjA_ref, adjB_ref, xfull_ref, w_ref, b_ref, xblk_ref, out_ref):
    accA = jnp.dot(adjA_ref[...].astype(jnp.bfloat16), xfull_ref[...],
                   preferred_element_type=jnp.float32)
    accB = jnp.dot(adjB_ref[...].astype(jnp.bfloat16), xfull_ref[...],
                   preferred_element_type=jnp.float32)
    yA = jnp.dot(accA, w_ref[...], preferred_element_type=jnp.float32)
    yB = jnp.dot(accB, w_ref[...], preferred_element_type=jnp.float32)
    xblk = xblk_ref[...]
    b = b_ref[...]
    out_ref[0:BM, :] = jnp.maximum(yA + xblk[0:BM, :] + b, 0.0)
    out_ref[BM:2 * BM, :] = jnp.maximum(yB + xblk[BM:2 * BM, :] + b, 0.0)


@jax.jit
def kernel(input, adj, W, b):
    x = input
    b2 = b.reshape(1, D)

    out = pl.pallas_call(
        _gcn_body,
        grid=(N // (2 * BM),),
        in_specs=[
            pl.BlockSpec((BM, N), lambda i: (2 * i, 0)),
            pl.BlockSpec((BM, N), lambda i: (2 * i + 1, 0)),
            pl.BlockSpec((N, D), lambda i: (0, 0)),
            pl.BlockSpec((D, D), lambda i: (0, 0)),
            pl.BlockSpec((1, D), lambda i: (0, 0)),
            pl.BlockSpec((2 * BM, D), lambda i: (i, 0)),
        ],
        out_specs=pl.BlockSpec((2 * BM, D), lambda i: (i, 0)),
        out_shape=jax.ShapeDtypeStruct((N, D), jnp.float32),
        compiler_params=pltpu.CompilerParams(
            dimension_semantics=("arbitrary",),
        ),
    )(adj, adj, x.astype(jnp.bfloat16), W, b2, x)

    return out


# speedup vs baseline: 1.0434x; 1.0434x over previous
import jax
import jax.numpy as jnp
from jax.experimental import pallas as pl
from jax.experimental.pallas import tpu as pltpu

N = 10000
D = 128
BM = 200


def _gcn_body(adjA_ref, adjB_ref, xfull_ref, w_ref, b_ref, out_ref):
    i = pl.program_id(0)
    accA = jnp.dot(adjA_ref[...], xfull_ref[...],
                   preferred_element_type=jnp.float32)
    accB = jnp.dot(adjB_ref[...], xfull_ref[...],
                   preferred_element_type=jnp.float32)
    yA = jnp.dot(accA, w_ref[...], preferred_element_type=jnp.float32)
    yB = jnp.dot(accB, w_ref[...], preferred_element_type=jnp.float32)
    b = b_ref[...]
    base = i * 2 * BM
    xa = xfull_ref[pl.ds(base, BM), :]
    xb = xfull_ref[pl.ds(base + BM, BM), :]
    out_ref[0:BM, :] = jnp.maximum(yA + xa + b, 0.0)
    out_ref[BM:2 * BM, :] = jnp.maximum(yB + xb + b, 0.0)


@jax.jit
def kernel(input, adj, W, b):
    x = input
    b2 = b.reshape(1, D)

    out = pl.pallas_call(
        _gcn_body,
        grid=(N // (2 * BM),),
        in_specs=[
            pl.BlockSpec((BM, N), lambda i: (2 * i, 0)),
            pl.BlockSpec((BM, N), lambda i: (2 * i + 1, 0)),
            pl.BlockSpec((N, D), lambda i: (0, 0)),
            pl.BlockSpec((D, D), lambda i: (0, 0)),
            pl.BlockSpec((1, D), lambda i: (0, 0)),
        ],
        out_specs=pl.BlockSpec((2 * BM, D), lambda i: (i, 0)),
        out_shape=jax.ShapeDtypeStruct((N, D), jnp.float32),
        compiler_params=pltpu.CompilerParams(
            dimension_semantics=("arbitrary",),
        ),
    )(adj, adj, x, W, b2)

    return out
